# Initial kernel scaffold; baseline (speedup 1.0000x reference)
import jax
import jax.numpy as jnp
from jax.experimental import pallas as pl


def kernel(text_feat, audio_feat, video_feat, subject_id, params):
    raise NotImplementedError("write your pallas kernel here")


# speedup vs baseline: 1.8598x; 1.8598x over previous
import jax
import jax.numpy as jnp
import numpy as np
from jax.experimental import pallas as pl
from jax.experimental.pallas import tpu as pltpu

D = 512; INTER = 768; E = 8; TOPK = 2; FF = 1024; NH = 8; HD = 64; NL = 4
NSUBJ = 25; NVERT = 5124; LR = 256; MAXSEQ = 2048
B = 4; T = 256
S = 3 * T
N = B * S
VBLK = 1024

f32 = jnp.float32
bf16 = jnp.bfloat16


def _xsum(x):
    w = x.shape[-1]
    acc = x[:, :128] if w > 128 else x
    for i in range(1, w // 128):
        acc = acc + x[:, 128 * i:128 * (i + 1)]
    w = acc.shape[-1]
    a8 = acc[:, :8]
    for i in range(1, w // 8):
        a8 = a8 + acc[:, 8 * i:8 * (i + 1)]
    ww = 8
    while ww > 1:
        a8 = a8[:, :ww // 2] + a8[:, ww // 2:]
        ww //= 2
    return a8


def _xmean(x):
    return _xsum(x) * np.float32(1.0 / x.shape[-1])


def _ln(x, g, b, eps=1e-5):
    m = _xmean(x)
    d = x - m
    v = _xmean(d * d)
    return d / jnp.sqrt(v + eps) * g + b


def _xsoftmax(x):
    m = jnp.max(x, axis=-1, keepdims=True)
    e = jnp.exp(x - m)
    return e / _xsum(e)


def _dot(a, b):
    return jnp.dot(a, b, preferred_element_type=f32)


def _gelu(x):
    return jax.nn.gelu(x, approximate=False)


def _full(shp):
    return pl.BlockSpec(shp, lambda *_: (0,) * len(shp))


def _pj1_body(x_ref, g1, bb1, w1, b1, h_ref):
    h = _ln(x_ref[0], g1[...], bb1[...])
    h_ref[0] = _dot(h.astype(bf16), w1[...]) + b1[...]


def _pjmm_body(x_ref, w_ref, b_ref, h_ref):
    h_ref[0] = _dot(x_ref[0].astype(bf16), w_ref[...]) + b_ref[...]


def _pj3_body(x_ref, w3, b3, g2, bb2, me, pos, out_ref):
    h = _dot(x_ref[0].astype(bf16), w3[...]) + b3[...]
    h = _ln(h, g2[...], bb2[...])
    out_ref[0] = (h + me[...]) + pos[...]


def _projector(x, p, me_row, pos_m):
    in_dim = x.shape[-1]
    bs = lambda w: pl.BlockSpec((1, T, w), lambda b: (b, 0, 0))
    h1 = pl.pallas_call(
        _pj1_body, grid=(B,),
        in_specs=[bs(in_dim), _full((1, in_dim)), _full((1, in_dim)),
                  _full((in_dim, INTER)), _full((1, INTER))],
        out_specs=bs(INTER),
        out_shape=jax.ShapeDtypeStruct((B, T, INTER), f32),
    )(x, p['ln1_g'].reshape(1, in_dim), p['ln1_b'].reshape(1, in_dim),
      p['w1'].astype(bf16), p['b1'].reshape(1, INTER))
    h1 = _gelu(h1)
    h2 = pl.pallas_call(
        _pjmm_body, grid=(B,),
        in_specs=[bs(INTER), _full((INTER, INTER)), _full((1, INTER))],
        out_specs=bs(INTER),
        out_shape=jax.ShapeDtypeStruct((B, T, INTER), f32),
    )(h1, p['w2'].astype(bf16), p['b2'].reshape(1, INTER))
    h2 = _gelu(h2)
    return pl.pallas_call(
        _pj3_body, grid=(B,),
        in_specs=[bs(INTER), _full((INTER, D)), _full((1, D)),
                  _full((1, D)), _full((1, D)), _full((1, D)), _full((T, D))],
        out_specs=bs(D),
        out_shape=jax.ShapeDtypeStruct((B, T, D), f32),
    )(h2, p['w3'].astype(bf16), p['b3'].reshape(1, D),
      p['ln2_g'].reshape(1, D), p['ln2_b'].reshape(1, D),
      me_row, pos_m)


def _qkv_body(x_ref, g, b, wq, wk, wv, bq, bk, bv, q_ref, k_ref, v_ref):
    n1 = _ln(x_ref[0], g[...], b[...]).astype(bf16)
    q_ref[0] = _dot(n1, wq[...]) + bq[...]
    k_ref[0] = _dot(n1, wk[...]) + bk[...]
    v_ref[0] = _dot(n1, wv[...]) + bv[...]


def _qkv(x, lp):
    sd = jax.ShapeDtypeStruct((B, S, D), f32)
    iw, ib = lp['in_w'], lp['in_b']
    bspec = pl.BlockSpec((1, S, D), lambda b: (b, 0, 0))
    return pl.pallas_call(
        _qkv_body, grid=(B,),
        in_specs=[bspec, _full((1, D)), _full((1, D)),
                  _full((D, D)), _full((D, D)), _full((D, D)),
                  _full((1, D)), _full((1, D)), _full((1, D))],
        out_specs=[bspec] * 3,
        out_shape=[sd, sd, sd],
    )(x, lp['n1_g'].reshape(1, D), lp['n1_b'].reshape(1, D),
      iw[:, :D].astype(bf16), iw[:, D:2 * D].astype(bf16),
      iw[:, 2 * D:].astype(bf16),
      ib[:D].reshape(1, D), ib[D:2 * D].reshape(1, D), ib[2 * D:].reshape(1, D))


def _attn_body(q_ref, k_ref, v_ref, o_ref):
    q = q_ref[0].astype(bf16)
    k = k_ref[0].astype(bf16)
    v = v_ref[0].astype(bf16)
    s = jax.lax.dot_general(q, k, (((1,), (1,)), ((), ())),
                            preferred_element_type=f32) / 8.0
    p = _xsoftmax(s)
    o_ref[0] = _dot(p.astype(bf16), v)


def _attn_core(q, k, v):
    spec = pl.BlockSpec((1, S, HD), lambda i: (i, 0, 0))
    return pl.pallas_call(
        _attn_body, grid=(B * NH,),
        in_specs=[spec, spec, spec],
        out_specs=spec,
        out_shape=jax.ShapeDtypeStruct((B * NH, S, HD), f32),
    )(q, k, v)


def _post_body(x_ref, o_ref, ow, ob, g, b, gw, y_ref, n2_ref, lg_ref):
    y = x_ref[0] + (_dot(o_ref[0].astype(bf16), ow[...]) + ob[...])
    n2 = _ln(y, g[...], b[...])
    y_ref[0] = y
    n2_ref[0] = n2
    lg_ref[0] = _dot(n2.astype(bf16), gw[...])


def _post(x, o, lp):
    bspec = pl.BlockSpec((1, S, D), lambda b: (b, 0, 0))
    return pl.pallas_call(
        _post_body, grid=(B,),
        in_specs=[bspec, bspec, _full((D, D)), _full((1, D)),
                  _full((1, D)), _full((1, D)), _full((D, E))],
        out_specs=[bspec, bspec, pl.BlockSpec((1, S, E), lambda b: (b, 0, 0))],
        out_shape=[jax.ShapeDtypeStruct((B, S, D), f32),
                   jax.ShapeDtypeStruct((B, S, D), f32),
                   jax.ShapeDtypeStruct((B, S, E), f32)],
    )(x, o, lp['out_w'].astype(bf16), lp['out_b'].reshape(1, D),
      lp['n2_g'].reshape(1, D), lp['n2_b'].reshape(1, D),
      lp['gate_w'].astype(bf16))


def _top2(lg):
    rows = lg.shape[0]
    iota = jax.lax.broadcasted_iota(jnp.int32, (rows, E), 1)
    m1 = jnp.max(lg, axis=-1, keepdims=True)
    i1 = jnp.min(jnp.where(lg == m1, iota, E), axis=-1, keepdims=True)
    lg2 = jnp.where(iota == i1, jnp.float32(-1e30), lg)
    m2 = jnp.max(lg2, axis=-1, keepdims=True)
    i2 = jnp.min(jnp.where(lg2 == m2, iota, E), axis=-1, keepdims=True)
    e2 = jnp.exp(m2 - m1)
    denom = 1.0 + e2
    rw1 = 1.0 / denom
    rw2 = e2 / denom
    return i1, i2, rw1, rw2


def _ffn1_body(n2_ref, lg_ref, ew1, eb1, h1_ref, h2_ref):
    n2b = n2_ref[0].astype(bf16)
    i1, i2, _, _ = _top2(lg_ref[0])
    h1 = jnp.zeros((S, FF), f32)
    h2 = jnp.zeros((S, FF), f32)
    for e in range(E):
        he = _dot(n2b, ew1[e]) + eb1[e]
        h1 = h1 + jnp.where(i1 == e, he, 0.0)
        h2 = h2 + jnp.where(i2 == e, he, 0.0)
    h1_ref[0] = h1
    h2_ref[0] = h2


def _ffn2_body(g1_ref, g2_ref, lg_ref, y_ref, ew2, eb2, out_ref):
    i1, i2, rw1, rw2 = _top2(lg_ref[0])
    g1 = g1_ref[0]
    g2 = g2_ref[0]
    acc = jnp.zeros((S, D), f32)
    for e in range(E):
        gm = jnp.where(i1 == e, g1, 0.0) + jnp.where(i2 == e, g2, 0.0)
        de = _dot(gm.astype(bf16), ew2[e]) + eb2[e]
        ce = jnp.where(i1 == e, rw1, 0.0) + jnp.where(i2 == e, rw2, 0.0)
        acc = acc + ce * de
    out_ref[0] = y_ref[0] + acc


def _moe_ffn(n2, lg, y, lp):
    bspec = pl.BlockSpec((1, S, D), lambda b: (b, 0, 0))
    fspec = pl.BlockSpec((1, S, FF), lambda b: (b, 0, 0))
    lspec = pl.BlockSpec((1, S, E), lambda b: (b, 0, 0))
    hsd = jax.ShapeDtypeStruct((B, S, FF), f32)
    h1, h2 = pl.pallas_call(
        _ffn1_body, grid=(B,),
        in_specs=[bspec, lspec, _full((E, D, FF)), _full((E, 1, FF))],
        out_specs=[fspec, fspec],
        out_shape=[hsd, hsd],
    )(n2, lg, lp['ew1'].astype(bf16), lp['eb1'].reshape(E, 1, FF))
    g1 = _gelu(h1)
    g2 = _gelu(h2)
    return pl.pallas_call(
        _ffn2_body, grid=(B,),
        in_specs=[fspec, fspec, lspec, bspec,
                  _full((E, FF, D)), _full((E, 1, D))],
        out_specs=bspec,
        out_shape=jax.ShapeDtypeStruct((B, S, D), f32),
    )(g1, g2, lg, y, lp['ew2'].astype(bf16), lp['eb2'].reshape(E, 1, D))


def _aux_body(lg_ref, aux_ref):
    lg = lg_ref[...]
    iota = jax.lax.broadcasted_iota(jnp.int32, (N, E), 1)
    m1 = jnp.max(lg, axis=-1, keepdims=True)
    i1 = jnp.min(jnp.where(lg == m1, iota, E), axis=-1, keepdims=True)
    pm = jnp.exp(lg - m1)
    probs = pm / jnp.sum(pm, axis=-1, keepdims=True)
    rpe = jnp.mean(probs, axis=0, keepdims=True)
    tpe = jnp.mean((iota == i1).astype(f32), axis=0, keepdims=True)
    aux = E * jnp.sum(tpe * rpe) + 0.001 * jnp.mean(lg * lg)
    aux_ref[...] = aux.reshape(1, 1)


def _aux(logits):
    return pl.pallas_call(
        _aux_body, grid=(1,),
        in_specs=[pl.BlockSpec((N, E), lambda i: (0, 0))],
        out_specs=pl.BlockSpec((1, 1), lambda i: (0, 0)),
        out_shape=jax.ShapeDtypeStruct((1, 1), f32),
    )(logits)


def _low_body(x_ref, g, b, lw, out_ref):
    n = _ln(x_ref[0], g[...], b[...])
    out_ref[0] = _dot(n.astype(bf16), lw[...])


def _low(x, norm_g, norm_b, lr_w):
    return pl.pallas_call(
        _low_body, grid=(B,),
        in_specs=[pl.BlockSpec((1, S, D), lambda b: (b, 0, 0)),
                  _full((1, D)), _full((1, D)), _full((D, LR))],
        out_specs=pl.BlockSpec((1, S, LR), lambda b: (b, 0, 0)),
        out_shape=jax.ShapeDtypeStruct((B, S, LR), f32),
    )(x, norm_g.reshape(1, D), norm_b.reshape(1, D), lr_w.astype(bf16))


def _head_body(sid_ref, low_ref, w_ref, bb_ref, out_ref):
    w = w_ref[0].astype(bf16)
    out_ref[0] = _dot(low_ref[0].astype(bf16), w) + bb_ref[0]


def _head(subject_id, low, subj_w, subj_b):
    nvb = pl.cdiv(NVERT, VBLK)
    grid_spec = pltpu.PrefetchScalarGridSpec(
        num_scalar_prefetch=1,
        grid=(B, nvb),
        in_specs=[
            pl.BlockSpec((1, S, LR), lambda b, v, sid: (b, 0, 0)),
            pl.BlockSpec((1, LR, VBLK), lambda b, v, sid: (sid[b], 0, v)),
            pl.BlockSpec((1, 1, VBLK), lambda b, v, sid: (sid[b], 0, v)),
        ],
        out_specs=pl.BlockSpec((1, S, VBLK), lambda b, v, sid: (b, 0, v)),
    )
    return pl.pallas_call(
        _head_body,
        grid_spec=grid_spec,
        out_shape=jax.ShapeDtypeStruct((B, S, NVERT), f32),
    )(subject_id, low, subj_w, subj_b)


def _to_heads(a):
    return a.reshape(B, S, NH, HD).transpose(0, 2, 1, 3).reshape(B * NH, S, HD)


def _jln(x, g, b, eps=1e-5):
    m = jnp.mean(x, axis=-1, keepdims=True)
    v = jnp.var(x, axis=-1, keepdims=True)
    return (x - m) / jnp.sqrt(v + eps) * g + b


def _xla_projector(x, p):
    x = _jln(x, p['ln1_g'], p['ln1_b'])
    x = _gelu(x @ p['w1'] + p['b1'])
    x = _gelu(x @ p['w2'] + p['b2'])
    x = x @ p['w3'] + p['b3']
    return _jln(x, p['ln2_g'], p['ln2_b'])


def kernel(text_feat, audio_feat, video_feat, subject_id, params):
    me = params['mod_embed']
    tp = _xla_projector(text_feat, params['text_proj']) + me[0]
    ap = _xla_projector(audio_feat, params['audio_proj']) + me[1]
    vp = _xla_projector(video_feat, params['video_proj']) + me[2]

    x = jnp.stack([tp, ap, vp], axis=2).reshape(B, S, D)
    x = x + params['pos_embed'][:, :S, :]

    aux_total = jnp.float32(0.0)
    for lp in params['layers']:
        q, k, v = _qkv(x, lp)
        o = _attn_core(_to_heads(q), _to_heads(k), _to_heads(v))
        o = o.reshape(B, NH, S, HD).transpose(0, 2, 1, 3).reshape(B, S, D)
        y, n2, lg = _post(x, o, lp)
        aux = _aux(lg.reshape(N, E))
        aux_total = aux_total + aux[0, 0]
        x = _moe_ffn(n2, lg, y, lp)

    low = _low(x, params['norm_g'], params['norm_b'], params['lr_w'])
    out = _head(subject_id, low, params['subj_w'], params['subj_b'])
    return out, aux_total * 0.01
